# Initial kernel scaffold; baseline (speedup 1.0000x reference)
import jax
import jax.numpy as jnp
from jax.experimental import pallas as pl


def kernel(x, table):
    raise NotImplementedError("write your pallas kernel here")


# speedup vs baseline: 1.1523x; 1.1523x over previous
import functools

import jax
import jax.numpy as jnp
from jax import lax
from jax.experimental import pallas as pl
from jax.experimental.pallas import tpu as pltpu
from jax.experimental.pallas import tpu_sc as plsc


@functools.lru_cache(maxsize=None)
def _make_sc_gather(V, D, B):
    info = plsc.get_sparse_core_info()
    NC, NS = info.num_cores, info.num_subcores
    NW = NC * NS
    assert B % NW == 0
    b_per_w = B // NW
    chunk = 1600
    assert b_per_w % chunk == 0
    n_chunks = b_per_w // chunk
    mesh = plsc.VectorSubcoreMesh(core_axis_name="c", subcore_axis_name="s")

    @functools.partial(
        pl.kernel,
        mesh=mesh,
        compiler_params=pltpu.CompilerParams(use_tc_tiling_on_sc=False),
        out_type=jax.ShapeDtypeStruct((B, D), jnp.float32),
        scratch_types=[
            pltpu.VMEM((chunk,), jnp.int32),
            pltpu.VMEM((chunk, D), jnp.float32),
            pltpu.SemaphoreType.DMA,
        ],
    )
    def k(table_hbm, idx_hbm, out_hbm, idx_v, rows_v, sem):
        wid = lax.axis_index("s") * NC + lax.axis_index("c")
        base = wid * b_per_w

        def body(i, carry):
            off = base + i * chunk
            pltpu.sync_copy(idx_hbm.at[pl.ds(off, chunk)], idx_v)
            pltpu.async_copy(table_hbm.at[idx_v], rows_v, sem).wait()
            pltpu.sync_copy(rows_v, out_hbm.at[pl.ds(off, chunk)])
            return carry

        lax.fori_loop(0, n_chunks, body, 0)

    return k


def kernel(x, table):
    S0, S1 = x.shape
    V, D = table.shape
    B = S0 * S1
    out = _make_sc_gather(V, D, B)(table, x.reshape(B))
    return out.reshape(S0, S1, D)


# speedup vs baseline: 1.1587x; 1.0055x over previous
import functools

import jax
import jax.numpy as jnp
from jax import lax
from jax.experimental import pallas as pl
from jax.experimental.pallas import tpu as pltpu
from jax.experimental.pallas import tpu_sc as plsc


@functools.lru_cache(maxsize=None)
def _make_sc_gather(V, D, B):
    info = plsc.get_sparse_core_info()
    NC, NS = info.num_cores, info.num_subcores
    NW = NC * NS
    assert B % NW == 0
    b_per_w = B // NW
    chunk = 1600
    assert b_per_w % chunk == 0
    n_chunks = b_per_w // chunk
    nbuf = 2
    mesh = plsc.VectorSubcoreMesh(core_axis_name="c", subcore_axis_name="s")

    @functools.partial(
        pl.kernel,
        mesh=mesh,
        compiler_params=pltpu.CompilerParams(use_tc_tiling_on_sc=False),
        out_type=jax.ShapeDtypeStruct((B, D), jnp.float32),
        scratch_types=(
            [pltpu.VMEM((chunk,), jnp.int32)] * nbuf
            + [pltpu.VMEM((chunk, D), jnp.float32)] * nbuf
            + [pltpu.SemaphoreType.DMA] * (2 * nbuf)
        ),
    )
    def k(table_hbm, idx_hbm, out_hbm, *scratch):
        idx_v = scratch[:nbuf]
        rows_v = scratch[nbuf : 2 * nbuf]
        gsem = scratch[2 * nbuf : 3 * nbuf]
        wsem = scratch[3 * nbuf :]
        wid = lax.axis_index("s") * NC + lax.axis_index("c")
        base = wid * b_per_w

        def start_gather(i, j):
            off = base + i * chunk
            pltpu.sync_copy(idx_hbm.at[pl.ds(off, chunk)], idx_v[j])
            return pltpu.async_copy(table_hbm.at[idx_v[j]], rows_v[j], gsem[j])

        g = [None] * nbuf
        w = [None] * nbuf
        g[0] = start_gather(0, 0)
        for i in range(n_chunks):
            j = i % nbuf
            nj = (i + 1) % nbuf
            if i + 1 < n_chunks:
                if w[nj] is not None:
                    w[nj].wait()
                    w[nj] = None
                g[nj] = start_gather(i + 1, nj)
            g[j].wait()
            off = base + i * chunk
            w[j] = pltpu.async_copy(rows_v[j], out_hbm.at[pl.ds(off, chunk)], wsem[j])
        for j in range(nbuf):
            if w[j] is not None:
                w[j].wait()

    return k


def kernel(x, table):
    S0, S1 = x.shape
    V, D = table.shape
    B = S0 * S1
    out = _make_sc_gather(V, D, B)(table, x.reshape(B))
    return out.reshape(S0, S1, D)


# speedup vs baseline: 1.8733x; 1.6168x over previous
import functools

import jax
import jax.numpy as jnp
from jax import lax
from jax.experimental import pallas as pl
from jax.experimental.pallas import tpu as pltpu
from jax.experimental.pallas import tpu_sc as plsc


@functools.lru_cache(maxsize=None)
def _make_sc_gather(V, D, S0, S1):
    info = plsc.get_sparse_core_info()
    NC, NS = info.num_cores, info.num_subcores
    NW = NC * NS
    B = S0 * S1
    assert B % NW == 0
    b_per_w = B // NW
    chunk_rows = 32
    chunk = chunk_rows * S1
    assert b_per_w % chunk == 0
    n_chunks = b_per_w // chunk
    rows_per_w = b_per_w // S1
    nbuf = 2
    mesh = plsc.VectorSubcoreMesh(core_axis_name="c", subcore_axis_name="s")

    @functools.partial(
        pl.kernel,
        mesh=mesh,
        compiler_params=pltpu.CompilerParams(use_tc_tiling_on_sc=False),
        out_type=jax.ShapeDtypeStruct((S0, S1, D), jnp.float32),
        scratch_types=(
            [pltpu.VMEM((chunk,), jnp.int32)] * nbuf
            + [pltpu.VMEM((chunk, D), jnp.float32)] * nbuf
            + [pltpu.SemaphoreType.DMA] * (2 * nbuf)
        ),
    )
    def k(table_hbm, x_hbm, out_hbm, *scratch):
        idx_v = scratch[:nbuf]
        rows_v = scratch[nbuf : 2 * nbuf]
        gsem = scratch[2 * nbuf : 3 * nbuf]
        wsem = scratch[3 * nbuf :]
        wid = lax.axis_index("s") * NC + lax.axis_index("c")
        row_base = wid * rows_per_w

        def start_gather(i, j):
            off = (row_base + i * chunk_rows) * S1
            pltpu.sync_copy(x_hbm.at[pl.ds(off, chunk)], idx_v[j])
            return pltpu.async_copy(table_hbm.at[idx_v[j]], rows_v[j], gsem[j])

        def start_write(i, j):
            r0 = row_base + i * chunk_rows
            last = None
            for a in range(chunk_rows):
                last = pltpu.async_copy(
                    rows_v[j].at[pl.ds(a * S1, S1)],
                    out_hbm.at[r0 + a],
                    wsem[j],
                )
            return last

        def drain_write(j):
            for a in range(chunk_rows):
                w[j].wait()

        g = [None] * nbuf
        w = [None] * nbuf
        g[0] = start_gather(0, 0)
        for i in range(n_chunks):
            j = i % nbuf
            nj = (i + 1) % nbuf
            if i + 1 < n_chunks:
                if w[nj] is not None:
                    drain_write(nj)
                    w[nj] = None
                g[nj] = start_gather(i + 1, nj)
            g[j].wait()
            w[j] = start_write(i, j)
        for j in range(nbuf):
            if w[j] is not None:
                drain_write(j)

    return k


def kernel(x, table):
    S0, S1 = x.shape
    V, D = table.shape
    return _make_sc_gather(V, D, S0, S1)(table, x.reshape(S0 * S1))


# speedup vs baseline: 1.8750x; 1.0009x over previous
import functools

import jax
import jax.numpy as jnp
from jax import lax
from jax.experimental import pallas as pl
from jax.experimental.pallas import tpu as pltpu
from jax.experimental.pallas import tpu_sc as plsc


@functools.lru_cache(maxsize=None)
def _make_sc_gather(V, D, S0, S1):
    info = plsc.get_sparse_core_info()
    NC, NS = info.num_cores, info.num_subcores
    NW = NC * NS
    B = S0 * S1
    assert B % NW == 0
    b_per_w = B // NW
    chunk_rows = 32
    chunk = chunk_rows * S1
    assert b_per_w % chunk == 0
    n_chunks = b_per_w // chunk
    rows_per_w = b_per_w // S1
    nbuf = 2
    mesh = plsc.VectorSubcoreMesh(core_axis_name="c", subcore_axis_name="s")

    @functools.partial(
        pl.kernel,
        mesh=mesh,
        compiler_params=pltpu.CompilerParams(
            use_tc_tiling_on_sc=False, needs_layout_passes=False
        ),
        out_type=jax.ShapeDtypeStruct((S0, S1, D), jnp.float32),
        scratch_types=(
            [pltpu.VMEM((chunk,), jnp.int32)] * nbuf
            + [pltpu.VMEM((chunk, D), jnp.float32)] * nbuf
            + [pltpu.SemaphoreType.DMA] * (2 * nbuf)
        ),
    )
    def k(table_hbm, x_hbm, out_hbm, *scratch):
        idx_v = scratch[:nbuf]
        rows_v = scratch[nbuf : 2 * nbuf]
        gsem = scratch[2 * nbuf : 3 * nbuf]
        wsem = scratch[3 * nbuf :]
        wid = lax.axis_index("s") * NC + lax.axis_index("c")
        row_base = wid * rows_per_w

        def start_gather(i, j):
            off = (row_base + i * chunk_rows) * S1
            pltpu.sync_copy(x_hbm.at[pl.ds(off, chunk)], idx_v[j])
            return pltpu.async_copy(table_hbm.at[idx_v[j]], rows_v[j], gsem[j])

        def start_write(i, j):
            r0 = row_base + i * chunk_rows
            last = None
            for a in range(chunk_rows):
                last = pltpu.async_copy(
                    rows_v[j].at[pl.ds(a * S1, S1)],
                    out_hbm.at[r0 + a],
                    wsem[j],
                )
            return last

        def drain_write(j):
            for a in range(chunk_rows):
                w[j].wait()

        g = [None] * nbuf
        w = [None] * nbuf
        g[0] = start_gather(0, 0)
        for i in range(n_chunks):
            j = i % nbuf
            nj = (i + 1) % nbuf
            if i + 1 < n_chunks:
                if w[nj] is not None:
                    drain_write(nj)
                    w[nj] = None
                g[nj] = start_gather(i + 1, nj)
            g[j].wait()
            w[j] = start_write(i, j)
        for j in range(nbuf):
            if w[j] is not None:
                drain_write(j)

    return k


def kernel(x, table):
    S0, S1 = x.shape
    V, D = table.shape
    return _make_sc_gather(V, D, S0, S1)(table, x.reshape(S0 * S1))
